# Initial kernel scaffold; baseline (speedup 1.0000x reference)
import jax
import jax.numpy as jnp
from jax.experimental import pallas as pl


def kernel(x, Wg, bg, W1, b1, W2, b2):
    raise NotImplementedError("write your pallas kernel here")


# speedup vs baseline: 4.0560x; 4.0560x over previous
import functools

import jax
import jax.numpy as jnp
from jax.experimental import pallas as pl
from jax.experimental.pallas import tpu as pltpu

_KTOP = 2
_NEG = float(jnp.finfo(jnp.float32).min)


def _moe_block(x_ref, wg_ref, bg_ref, w1_ref, b1_ref, w2_ref, b2_ref, out_ref):
    x = x_ref[...]
    E = wg_ref.shape[-1]
    H = b1_ref.shape[-1] // E

    logits = jnp.dot(x, wg_ref[...], preferred_element_type=jnp.float32)
    logits = logits + bg_ref[...]
    eidx = jax.lax.broadcasted_iota(jnp.int32, logits.shape, 1)
    m1 = jnp.max(logits, axis=-1, keepdims=True)
    i1 = jnp.min(jnp.where(logits == m1, eidx, E), axis=-1, keepdims=True)
    l2 = jnp.where(eidx == i1, _NEG, logits)
    m2 = jnp.max(l2, axis=-1, keepdims=True)
    i2 = jnp.min(jnp.where(l2 == m2, eidx, E), axis=-1, keepdims=True)
    mask = (eidx == i1) | (eidx == i2)
    p = jnp.exp(logits - m1)
    pm = jnp.where(mask, p, 0.0)
    g = pm / jnp.sum(pm, axis=-1, keepdims=True)

    h = jnp.dot(x, w1_ref[...], preferred_element_type=jnp.float32)
    h = h + b1_ref[...]
    h = 0.5 * h * (1.0 + jax.lax.erf(h * 0.7071067811865476))
    sel_col = jax.lax.broadcasted_iota(jnp.int32, (E, E * H), 1) // H
    sel_row = jax.lax.broadcasted_iota(jnp.int32, (E, E * H), 0)
    sel = (sel_col == sel_row).astype(jnp.float32)
    g_exp = jnp.dot(g, sel, preferred_element_type=jnp.float32)
    hg = h * g_exp
    out = jnp.dot(hg, w2_ref[...], preferred_element_type=jnp.float32)
    out = out + jnp.dot(g, b2_ref[...], preferred_element_type=jnp.float32)
    out_ref[...] = out


def kernel(x, Wg, bg, W1, b1, W2, b2):
    B, D = x.shape
    E = Wg.shape[-1]
    H = W1.shape[-1]
    w1_all = jnp.transpose(W1, (1, 0, 2)).reshape(D, E * H)
    b1_all = b1.reshape(1, E * H)
    w2_all = W2.reshape(E * H, D)
    bg2 = bg.reshape(1, E)

    Bb = 1024
    grid = (B // Bb,)
    const = lambda i: (0, 0)
    out = pl.pallas_call(
        _moe_block,
        grid=grid,
        in_specs=[
            pl.BlockSpec((Bb, D), lambda i: (i, 0)),
            pl.BlockSpec((D, E), const),
            pl.BlockSpec((1, E), const),
            pl.BlockSpec((D, E * H), const),
            pl.BlockSpec((1, E * H), const),
            pl.BlockSpec((E * H, D), const),
            pl.BlockSpec((E, D), const),
        ],
        out_specs=pl.BlockSpec((Bb, D), lambda i: (i, 0)),
        out_shape=jax.ShapeDtypeStruct((B, D), jnp.float32),
        compiler_params=pltpu.CompilerParams(
            dimension_semantics=("arbitrary",),
        ),
    )(x, Wg, bg2, w1_all, b1_all, w2_all, b2)
    return out


# speedup vs baseline: 4.2529x; 1.0485x over previous
import functools

import jax
import jax.numpy as jnp
from jax.experimental import pallas as pl
from jax.experimental.pallas import tpu as pltpu

_KTOP = 2
_NEG = float(jnp.finfo(jnp.float32).min)


def _moe_block(x_ref, wg_ref, bg_ref, w1_ref, b1_ref, w2_ref, b2_ref, out_ref):
    x = x_ref[...]
    E = wg_ref.shape[-1]
    H = b1_ref.shape[-1] // E

    logits = jnp.dot(x, wg_ref[...], preferred_element_type=jnp.float32)
    logits = logits + bg_ref[...]
    eidx = jax.lax.broadcasted_iota(jnp.int32, logits.shape, 1)
    m1 = jnp.max(logits, axis=-1, keepdims=True)
    i1 = jnp.min(jnp.where(logits == m1, eidx, E), axis=-1, keepdims=True)
    l2 = jnp.where(eidx == i1, _NEG, logits)
    m2 = jnp.max(l2, axis=-1, keepdims=True)
    i2 = jnp.min(jnp.where(l2 == m2, eidx, E), axis=-1, keepdims=True)
    mask = (eidx == i1) | (eidx == i2)
    p = jnp.exp(logits - m1)
    pm = jnp.where(mask, p, 0.0)
    g = pm / jnp.sum(pm, axis=-1, keepdims=True)

    xb = x.astype(jnp.bfloat16)
    h = jnp.dot(xb, w1_ref[...].astype(jnp.bfloat16),
                preferred_element_type=jnp.float32)
    h = h + b1_ref[...]
    h = 0.5 * h * (1.0 + jax.lax.erf(h * 0.7071067811865476))
    sel_col = jax.lax.broadcasted_iota(jnp.int32, (E, E * H), 1) // H
    sel_row = jax.lax.broadcasted_iota(jnp.int32, (E, E * H), 0)
    sel = (sel_col == sel_row).astype(jnp.float32)
    g_exp = jnp.dot(g, sel, preferred_element_type=jnp.float32)
    hg = (h * g_exp).astype(jnp.bfloat16)
    out = jnp.dot(hg, w2_ref[...].astype(jnp.bfloat16),
                  preferred_element_type=jnp.float32)
    out = out + jnp.dot(g, b2_ref[...], preferred_element_type=jnp.float32)
    out_ref[...] = out


def kernel(x, Wg, bg, W1, b1, W2, b2):
    B, D = x.shape
    E = Wg.shape[-1]
    H = W1.shape[-1]
    w1_all = jnp.transpose(W1, (1, 0, 2)).reshape(D, E * H)
    b1_all = b1.reshape(1, E * H)
    w2_all = W2.reshape(E * H, D)
    bg2 = bg.reshape(1, E)

    Bb = 1024
    grid = (B // Bb,)
    const = lambda i: (0, 0)
    out = pl.pallas_call(
        _moe_block,
        grid=grid,
        in_specs=[
            pl.BlockSpec((Bb, D), lambda i: (i, 0)),
            pl.BlockSpec((D, E), const),
            pl.BlockSpec((1, E), const),
            pl.BlockSpec((D, E * H), const),
            pl.BlockSpec((1, E * H), const),
            pl.BlockSpec((E * H, D), const),
            pl.BlockSpec((E, D), const),
        ],
        out_specs=pl.BlockSpec((Bb, D), lambda i: (i, 0)),
        out_shape=jax.ShapeDtypeStruct((B, D), jnp.float32),
        compiler_params=pltpu.CompilerParams(
            dimension_semantics=("arbitrary",),
        ),
    )(x, Wg, bg2, w1_all, b1_all, w2_all, b2)
    return out
